# baseline (device time: 55515 ns/iter reference)
import jax
import jax.numpy as jnp
from jax import lax
from jax.experimental import pallas as pl
from jax.experimental.pallas import tpu as pltpu


def kernel(A, B):
    M, Ks = A.shape
    _, N = B.shape

    def body(a_ref, b_ref, out_ref, comm_ref, send_sem, recv_sem):
        my_x = lax.axis_index("x")
        my_y = lax.axis_index("y")
        peer = (1 - my_x, my_y)

        barrier_sem = pltpu.get_barrier_semaphore()
        pl.semaphore_signal(
            barrier_sem, inc=1, device_id=peer,
            device_id_type=pl.DeviceIdType.MESH,
        )
        pl.semaphore_wait(barrier_sem, 1)

        out_ref[:, :] = jnp.dot(
            a_ref[:, :], b_ref[:, :], preferred_element_type=jnp.float32
        )

        rdma = pltpu.make_async_remote_copy(
            src_ref=out_ref,
            dst_ref=comm_ref,
            send_sem=send_sem,
            recv_sem=recv_sem,
            device_id=peer,
            device_id_type=pl.DeviceIdType.MESH,
        )
        rdma.start()
        rdma.wait()

        out_ref[:, :] = out_ref[:, :] + comm_ref[:, :]

    return pl.pallas_call(
        body,
        out_shape=jax.ShapeDtypeStruct((M, N), jnp.float32),
        in_specs=[
            pl.BlockSpec(memory_space=pltpu.VMEM),
            pl.BlockSpec(memory_space=pltpu.VMEM),
        ],
        out_specs=pl.BlockSpec(memory_space=pltpu.VMEM),
        scratch_shapes=[
            pltpu.VMEM((M, N), jnp.float32),
            pltpu.SemaphoreType.DMA,
            pltpu.SemaphoreType.DMA,
        ],
        compiler_params=pltpu.CompilerParams(collective_id=0),
    )(A, B)


# device time: 39789 ns/iter; 1.3952x vs baseline; 1.3952x over previous
import jax
import jax.numpy as jnp
from jax import lax
from jax.experimental import pallas as pl
from jax.experimental.pallas import tpu as pltpu

N_CHUNKS = 4


def kernel(A, B):
    M, Ks = A.shape
    _, N = B.shape
    HALF = M // 2
    CH = HALF // N_CHUNKS

    def body(a_ref, b_ref, out_ref, xrecv_buf,
             xsend_sems, xrecv_sems, ysend_sems, yrecv_sems):
        my_x = lax.axis_index("x")
        my_y = lax.axis_index("y")
        xpeer = (1 - my_x, my_y)
        ypeer = (my_x, 1 - my_y)
        base = my_y * HALF

        barrier_sem = pltpu.get_barrier_semaphore()
        for peer in (xpeer, ypeer):
            pl.semaphore_signal(
                barrier_sem, inc=1, device_id=peer,
                device_id_type=pl.DeviceIdType.MESH,
            )
        pl.semaphore_wait(barrier_sem, 2)

        def x_rdma(c):
            rows = pl.ds(base + c * CH, CH)
            return pltpu.make_async_remote_copy(
                src_ref=out_ref.at[rows, :],
                dst_ref=xrecv_buf.at[c],
                send_sem=xsend_sems.at[c],
                recv_sem=xrecv_sems.at[c],
                device_id=xpeer,
                device_id_type=pl.DeviceIdType.MESH,
            )

        def y_rdma(c):
            rows = pl.ds(base + c * CH, CH)
            return pltpu.make_async_remote_copy(
                src_ref=out_ref.at[rows, :],
                dst_ref=out_ref.at[rows, :],
                send_sem=ysend_sems.at[c],
                recv_sem=yrecv_sems.at[c],
                device_id=ypeer,
                device_id_type=pl.DeviceIdType.MESH,
            )

        for c in range(N_CHUNKS):
            rows = pl.ds(base + c * CH, CH)
            out_ref[rows, :] = jnp.dot(
                a_ref[rows, :], b_ref[:, :],
                preferred_element_type=jnp.float32,
            )
            x_rdma(c).start()

        for c in range(N_CHUNKS):
            r = x_rdma(c)
            r.wait_send()
            r.wait_recv()
            rows = pl.ds(base + c * CH, CH)
            out_ref[rows, :] = out_ref[rows, :] + xrecv_buf[c]
            y_rdma(c).start()

        for c in range(N_CHUNKS):
            y_rdma(c).wait_send()
            y_rdma(c).wait_recv()

    return pl.pallas_call(
        body,
        out_shape=jax.ShapeDtypeStruct((M, N), jnp.float32),
        in_specs=[
            pl.BlockSpec(memory_space=pltpu.VMEM),
            pl.BlockSpec(memory_space=pltpu.VMEM),
        ],
        out_specs=pl.BlockSpec(memory_space=pltpu.VMEM),
        scratch_shapes=[
            pltpu.VMEM((N_CHUNKS, CH, N), jnp.float32),
            pltpu.SemaphoreType.DMA((N_CHUNKS,)),
            pltpu.SemaphoreType.DMA((N_CHUNKS,)),
            pltpu.SemaphoreType.DMA((N_CHUNKS,)),
            pltpu.SemaphoreType.DMA((N_CHUNKS,)),
        ],
        compiler_params=pltpu.CompilerParams(collective_id=0),
    )(A, B)


# device time: 37058 ns/iter; 1.4981x vs baseline; 1.0737x over previous
import jax
import jax.numpy as jnp
from jax import lax
from jax.experimental import pallas as pl
from jax.experimental.pallas import tpu as pltpu

N_CHUNKS = 8


def kernel(A, B):
    M, Ks = A.shape
    _, N = B.shape
    HALF = M // 2
    CH = HALF // N_CHUNKS

    def body(a_ref, b_ref, out_ref, xrecv_buf,
             xsend_sems, xrecv_sems, ysend_sems, yrecv_sems):
        my_x = lax.axis_index("x")
        my_y = lax.axis_index("y")
        xpeer = (1 - my_x, my_y)
        ypeer = (my_x, 1 - my_y)
        base = my_y * HALF

        barrier_sem = pltpu.get_barrier_semaphore()
        for peer in (xpeer, ypeer):
            pl.semaphore_signal(
                barrier_sem, inc=1, device_id=peer,
                device_id_type=pl.DeviceIdType.MESH,
            )
        pl.semaphore_wait(barrier_sem, 2)

        def x_rdma(c):
            rows = pl.ds(base + c * CH, CH)
            return pltpu.make_async_remote_copy(
                src_ref=out_ref.at[rows, :],
                dst_ref=xrecv_buf.at[c],
                send_sem=xsend_sems.at[c],
                recv_sem=xrecv_sems.at[c],
                device_id=xpeer,
                device_id_type=pl.DeviceIdType.MESH,
            )

        def y_rdma(c):
            rows = pl.ds(base + c * CH, CH)
            return pltpu.make_async_remote_copy(
                src_ref=out_ref.at[rows, :],
                dst_ref=out_ref.at[rows, :],
                send_sem=ysend_sems.at[c],
                recv_sem=yrecv_sems.at[c],
                device_id=ypeer,
                device_id_type=pl.DeviceIdType.MESH,
            )

        for c in range(N_CHUNKS):
            rows = pl.ds(base + c * CH, CH)
            out_ref[rows, :] = jnp.dot(
                a_ref[rows, :], b_ref[:, :],
                preferred_element_type=jnp.float32,
            )
            x_rdma(c).start()

        for c in range(N_CHUNKS):
            r = x_rdma(c)
            r.wait_send()
            r.wait_recv()
            rows = pl.ds(base + c * CH, CH)
            out_ref[rows, :] = out_ref[rows, :] + xrecv_buf[c]
            y_rdma(c).start()

        for c in range(N_CHUNKS):
            y_rdma(c).wait_send()
            y_rdma(c).wait_recv()

    return pl.pallas_call(
        body,
        out_shape=jax.ShapeDtypeStruct((M, N), jnp.float32),
        in_specs=[
            pl.BlockSpec(memory_space=pltpu.VMEM),
            pl.BlockSpec(memory_space=pltpu.VMEM),
        ],
        out_specs=pl.BlockSpec(memory_space=pltpu.VMEM),
        scratch_shapes=[
            pltpu.VMEM((N_CHUNKS, CH, N), jnp.float32),
            pltpu.SemaphoreType.DMA((N_CHUNKS,)),
            pltpu.SemaphoreType.DMA((N_CHUNKS,)),
            pltpu.SemaphoreType.DMA((N_CHUNKS,)),
            pltpu.SemaphoreType.DMA((N_CHUNKS,)),
        ],
        compiler_params=pltpu.CompilerParams(collective_id=0),
    )(A, B)


# device time: 24468 ns/iter; 2.2689x vs baseline; 1.5145x over previous
import jax
import jax.numpy as jnp
from jax import lax
from jax.experimental import pallas as pl
from jax.experimental.pallas import tpu as pltpu

N_CHUNKS = 8


def kernel(A, B):
    M, Ks = A.shape
    _, N = B.shape
    HALF = M // 2
    CH = HALF // N_CHUNKS

    def body(a_ref, b_ref, out_ref, b_bf, xsend_buf, xrecv_buf,
             ysend_buf, yrecv_buf,
             xsend_sems, xrecv_sems, ysend_sems, yrecv_sems):
        my_x = lax.axis_index("x")
        my_y = lax.axis_index("y")
        xpeer = (1 - my_x, my_y)
        ypeer = (my_x, 1 - my_y)
        base = my_y * HALF
        obase = (1 - my_y) * HALF

        barrier_sem = pltpu.get_barrier_semaphore()
        for peer in (xpeer, ypeer):
            pl.semaphore_signal(
                barrier_sem, inc=1, device_id=peer,
                device_id_type=pl.DeviceIdType.MESH,
            )
        pl.semaphore_wait(barrier_sem, 2)

        def x_rdma(c):
            return pltpu.make_async_remote_copy(
                src_ref=xsend_buf.at[c],
                dst_ref=xrecv_buf.at[c],
                send_sem=xsend_sems.at[c],
                recv_sem=xrecv_sems.at[c],
                device_id=xpeer,
                device_id_type=pl.DeviceIdType.MESH,
            )

        def y_rdma(c):
            return pltpu.make_async_remote_copy(
                src_ref=ysend_buf.at[c],
                dst_ref=yrecv_buf.at[c],
                send_sem=ysend_sems.at[c],
                recv_sem=yrecv_sems.at[c],
                device_id=ypeer,
                device_id_type=pl.DeviceIdType.MESH,
            )

        b_bf[:, :] = b_ref[:, :].astype(jnp.bfloat16)

        for c in range(N_CHUNKS):
            rows = pl.ds(base + c * CH, CH)
            p = jnp.dot(
                a_ref[rows, :].astype(jnp.bfloat16), b_bf[:, :],
                preferred_element_type=jnp.float32,
            )
            out_ref[rows, :] = p
            xsend_buf[c] = p.astype(jnp.bfloat16)
            x_rdma(c).start()

        for c in range(N_CHUNKS):
            x_rdma(c).wait_recv()
            rows = pl.ds(base + c * CH, CH)
            red = out_ref[rows, :] + xrecv_buf[c].astype(jnp.float32)
            out_ref[rows, :] = red
            ysend_buf[c] = red.astype(jnp.bfloat16)
            y_rdma(c).start()

        for c in range(N_CHUNKS):
            y_rdma(c).wait_recv()
            orows = pl.ds(obase + c * CH, CH)
            out_ref[orows, :] = yrecv_buf[c].astype(jnp.float32)
        for c in range(N_CHUNKS):
            x_rdma(c).wait_send()
            y_rdma(c).wait_send()

    return pl.pallas_call(
        body,
        out_shape=jax.ShapeDtypeStruct((M, N), jnp.float32),
        in_specs=[
            pl.BlockSpec(memory_space=pltpu.VMEM),
            pl.BlockSpec(memory_space=pltpu.VMEM),
        ],
        out_specs=pl.BlockSpec(memory_space=pltpu.VMEM),
        scratch_shapes=[
            pltpu.VMEM((Ks, N), jnp.bfloat16),
            pltpu.VMEM((N_CHUNKS, CH, N), jnp.bfloat16),
            pltpu.VMEM((N_CHUNKS, CH, N), jnp.bfloat16),
            pltpu.VMEM((N_CHUNKS, CH, N), jnp.bfloat16),
            pltpu.VMEM((N_CHUNKS, CH, N), jnp.bfloat16),
            pltpu.SemaphoreType.DMA((N_CHUNKS,)),
            pltpu.SemaphoreType.DMA((N_CHUNKS,)),
            pltpu.SemaphoreType.DMA((N_CHUNKS,)),
            pltpu.SemaphoreType.DMA((N_CHUNKS,)),
        ],
        compiler_params=pltpu.CompilerParams(collective_id=0),
    )(A, B)
